# baseline (speedup 1.0000x reference)
import functools

import jax
import jax.numpy as jnp
from jax import lax
from jax.experimental import pallas as pl
from jax.experimental.pallas import tpu as pltpu
from jax.experimental.pallas import tpu_sc as plsc

VOCAB = 33
EMB = 128
B = 16384
L = 20
N = B * L
NUM_TABLES = 7
NUM_TC = 3
NUM_SC = NUM_TABLES - NUM_TC
VOCAB_PAD = 40

NC = 2
NS = 16
NW = NC * NS
BPW = B // NW
CB = 4
CHUNK = CB * L
NCHUNK = BPW // CB
NBUF = 4
LAG = 2
NROUND = NCHUNK // NBUF

L_PAD = 24
TC_BB = 128


def _sc_lookup(idx_all, tab_all):
    mesh = plsc.VectorSubcoreMesh(core_axis_name="c", subcore_axis_name="s")
    out_type = tuple(
        jax.ShapeDtypeStruct((B, L, EMB), jnp.float32)
        for _ in range(NUM_SC)
    )

    scratch = [pltpu.VMEM((NCHUNK, CHUNK), jnp.int32) for _ in range(2)]
    scratch += [pltpu.VMEM((CHUNK, EMB), jnp.float32) for _ in range(NBUF)]
    scratch += [pltpu.SemaphoreType.DMA for _ in range(2 + 2 * NBUF)]

    @functools.partial(
        pl.kernel,
        out_type=out_type,
        mesh=mesh,
        scratch_types=scratch,
    )
    def body(idx_ref, tab_ref, *refs):
        out_refs = refs[0:NUM_SC]
        rest = refs[NUM_SC:]
        idx_v = rest[0:2]
        rows = rest[2:2 + NBUF]
        isem = rest[2 + NBUF:4 + NBUF]
        gsem = rest[4 + NBUF:4 + 2 * NBUF]
        osem = rest[4 + 2 * NBUF:4 + 3 * NBUF]

        wid = lax.axis_index("s") * NC + lax.axis_index("c")
        bbase = wid * BPW
        ibase = wid * NCHUNK

        def start_idx(t):
            return pltpu.async_copy(
                idx_ref.at[t].at[pl.ds(ibase, NCHUNK)], idx_v[t % 2],
                isem[t % 2])

        def wait_idx(t):
            pltpu.make_async_copy(
                idx_ref.at[t].at[pl.ds(ibase, NCHUNK)], idx_v[t % 2],
                isem[t % 2]).wait()

        def gather_src(t, c):
            return tab_ref.at[t].at[pl.ds(wid * VOCAB_PAD, VOCAB_PAD)].at[
                idx_v[t % 2].at[c]]

        def start_gather(t, c, b):
            return pltpu.async_copy(gather_src(t, c), rows[b], gsem[b])

        def wait_gather(t, c, b):
            pltpu.make_async_copy(gather_src(t, c), rows[b], gsem[b]).wait()

        def start_out(t, c, b):
            pltpu.async_copy(rows[b].reshape(CB, L, EMB),
                             out_refs[t].at[pl.ds(bbase + c * CB, CB)],
                             osem[b])

        def wait_out(t, c, b):
            pltpu.make_async_copy(rows[b].reshape(CB, L, EMB),
                                  out_refs[t].at[pl.ds(bbase + c * CB, CB)],
                                  osem[b]).wait()

        start_idx(0)
        for t in range(NUM_SC):
            wait_idx(t)
            if t + 1 < NUM_SC:
                start_idx(t + 1)

            for b in range(NBUF):
                if t > 0:
                    wait_out(t - 1, NCHUNK - NBUF + b, b)
                start_gather(t, b, b)
                if b >= LAG:
                    bo = b - LAG
                    wait_gather(t, bo, bo)
                    start_out(t, bo, bo)

            @pl.loop(1, NROUND)
            def _(r, _t=t):
                for b in range(NBUF):
                    s = r * NBUF + b
                    wait_out(_t, s - NBUF, b)
                    start_gather(_t, s, b)
                    bo = (b - LAG) % NBUF
                    wait_gather(_t, s - LAG, bo)
                    start_out(_t, s - LAG, bo)

            for i in range(LAG):
                c = NCHUNK - LAG + i
                b = c % NBUF
                wait_gather(t, c, b)
                start_out(t, c, b)

        for b in range(NBUF):
            wait_out(NUM_SC - 1, NCHUNK - NBUF + b, b)

    return body(idx_all, tab_all)


def _tc_lookup(idx_flat_pad, tab):
    blk = TC_BB * L_PAD

    def tck(idx_ref, tab_ref, out_ref):
        v = idx_ref[...]
        iota = lax.broadcasted_iota(jnp.int32, (1, VOCAB), 1)
        onehot = (v[:, None] == iota).astype(jnp.float32)
        res = jnp.dot(onehot, tab_ref[...],
                      precision=lax.Precision.HIGHEST,
                      preferred_element_type=jnp.float32)
        res3 = res.reshape(TC_BB, L_PAD, EMB)
        out_ref[...] = res3[:, :L, :]

    return pl.pallas_call(
        tck,
        grid=(B // TC_BB,),
        in_specs=[
            pl.BlockSpec((blk,), lambda i: (i,)),
            pl.BlockSpec((VOCAB, EMB), lambda i: (0, 0)),
        ],
        out_specs=pl.BlockSpec((TC_BB, L, EMB), lambda i: (i, 0, 0)),
        out_shape=jax.ShapeDtypeStruct((B, L, EMB), jnp.float32),
    )(idx_flat_pad, tab)


def kernel(A1, A2, A3, B1, B2, B3, peptide,
           W_a1, W_a2, W_a3, W_b1, W_b2, W_b3, W_peptide):
    idx_in = (A1, A2, A3, B1, B2, B3, peptide)
    tab_in = (W_a1, W_a2, W_a3, W_b1, W_b2, W_b3, W_peptide)

    idx_all = jnp.stack(
        [x.reshape(N // CHUNK, CHUNK).astype(jnp.int32)
         for x in idx_in[:NUM_SC]])
    tab_all = jnp.stack(
        [jnp.tile(jnp.pad(w, ((0, VOCAB_PAD - VOCAB), (0, 0))), (NW, 1))
         for w in tab_in[:NUM_SC]])
    sc_outs = _sc_lookup(idx_all, tab_all)

    tc_outs = tuple(
        _tc_lookup(
            jnp.pad(x.astype(jnp.int32), ((0, 0), (0, L_PAD - L))
                    ).reshape(B * L_PAD),
            w)
        for x, w in zip(idx_in[NUM_SC:], tab_in[NUM_SC:]))

    return sc_outs + tc_outs


# speedup vs baseline: 1.0180x; 1.0180x over previous
import functools

import jax
import jax.numpy as jnp
from jax import lax
from jax.experimental import pallas as pl
from jax.experimental.pallas import tpu as pltpu
from jax.experimental.pallas import tpu_sc as plsc

VOCAB = 33
EMB = 128
B = 16384
L = 20
N = B * L
NUM_TABLES = 7
NUM_TC = 2
NUM_SC = NUM_TABLES - NUM_TC
VOCAB_PAD = 40

NC = 2
NS = 16
NW = NC * NS
BPW = B // NW
CB = 4
CHUNK = CB * L
NCHUNK = BPW // CB
NBUF = 4
LAG = 2
NROUND = NCHUNK // NBUF

L_PAD = 24
TC_BB = 128


def _sc_lookup(idx_all, tab_all):
    mesh = plsc.VectorSubcoreMesh(core_axis_name="c", subcore_axis_name="s")
    out_type = tuple(
        jax.ShapeDtypeStruct((B, L, EMB), jnp.float32)
        for _ in range(NUM_SC)
    )

    scratch = [pltpu.VMEM((NCHUNK, CHUNK), jnp.int32) for _ in range(2)]
    scratch += [pltpu.VMEM((CHUNK, EMB), jnp.float32) for _ in range(NBUF)]
    scratch += [pltpu.SemaphoreType.DMA for _ in range(2 + 2 * NBUF)]

    @functools.partial(
        pl.kernel,
        out_type=out_type,
        mesh=mesh,
        scratch_types=scratch,
    )
    def body(idx_ref, tab_ref, *refs):
        out_refs = refs[0:NUM_SC]
        rest = refs[NUM_SC:]
        idx_v = rest[0:2]
        rows = rest[2:2 + NBUF]
        isem = rest[2 + NBUF:4 + NBUF]
        gsem = rest[4 + NBUF:4 + 2 * NBUF]
        osem = rest[4 + 2 * NBUF:4 + 3 * NBUF]

        wid = lax.axis_index("s") * NC + lax.axis_index("c")
        bbase = wid * BPW
        ibase = wid * NCHUNK

        def start_idx(t):
            return pltpu.async_copy(
                idx_ref.at[t].at[pl.ds(ibase, NCHUNK)], idx_v[t % 2],
                isem[t % 2])

        def wait_idx(t):
            pltpu.make_async_copy(
                idx_ref.at[t].at[pl.ds(ibase, NCHUNK)], idx_v[t % 2],
                isem[t % 2]).wait()

        def gather_src(t, c):
            return tab_ref.at[t].at[pl.ds(wid * VOCAB_PAD, VOCAB_PAD)].at[
                idx_v[t % 2].at[c]]

        def start_gather(t, c, b):
            return pltpu.async_copy(gather_src(t, c), rows[b], gsem[b])

        def wait_gather(t, c, b):
            pltpu.make_async_copy(gather_src(t, c), rows[b], gsem[b]).wait()

        def start_out(t, c, b):
            pltpu.async_copy(rows[b].reshape(CB, L, EMB),
                             out_refs[t].at[pl.ds(bbase + c * CB, CB)],
                             osem[b])

        def wait_out(t, c, b):
            pltpu.make_async_copy(rows[b].reshape(CB, L, EMB),
                                  out_refs[t].at[pl.ds(bbase + c * CB, CB)],
                                  osem[b]).wait()

        start_idx(0)
        for t in range(NUM_SC):
            wait_idx(t)
            if t + 1 < NUM_SC:
                start_idx(t + 1)

            for b in range(NBUF):
                if t > 0:
                    wait_out(t - 1, NCHUNK - NBUF + b, b)
                start_gather(t, b, b)
                if b >= LAG:
                    bo = b - LAG
                    wait_gather(t, bo, bo)
                    start_out(t, bo, bo)

            @pl.loop(1, NROUND)
            def _(r, _t=t):
                for b in range(NBUF):
                    s = r * NBUF + b
                    wait_out(_t, s - NBUF, b)
                    start_gather(_t, s, b)
                    bo = (b - LAG) % NBUF
                    wait_gather(_t, s - LAG, bo)
                    start_out(_t, s - LAG, bo)

            for i in range(LAG):
                c = NCHUNK - LAG + i
                b = c % NBUF
                wait_gather(t, c, b)
                start_out(t, c, b)

        for b in range(NBUF):
            wait_out(NUM_SC - 1, NCHUNK - NBUF + b, b)

    return body(idx_all, tab_all)


def _tc_lookup(idx_flat_pad, tab):
    blk = TC_BB * L_PAD

    def tck(idx_ref, tab_ref, out_ref):
        v = idx_ref[...]
        iota = lax.broadcasted_iota(jnp.int32, (1, VOCAB), 1)
        onehot = (v[:, None] == iota).astype(jnp.float32)
        res = jnp.dot(onehot, tab_ref[...],
                      precision=lax.Precision.HIGHEST,
                      preferred_element_type=jnp.float32)
        res3 = res.reshape(TC_BB, L_PAD, EMB)
        out_ref[...] = res3[:, :L, :]

    return pl.pallas_call(
        tck,
        grid=(B // TC_BB,),
        in_specs=[
            pl.BlockSpec((blk,), lambda i: (i,)),
            pl.BlockSpec((VOCAB, EMB), lambda i: (0, 0)),
        ],
        out_specs=pl.BlockSpec((TC_BB, L, EMB), lambda i: (i, 0, 0)),
        out_shape=jax.ShapeDtypeStruct((B, L, EMB), jnp.float32),
    )(idx_flat_pad, tab)


def kernel(A1, A2, A3, B1, B2, B3, peptide,
           W_a1, W_a2, W_a3, W_b1, W_b2, W_b3, W_peptide):
    idx_in = (A1, A2, A3, B1, B2, B3, peptide)
    tab_in = (W_a1, W_a2, W_a3, W_b1, W_b2, W_b3, W_peptide)

    idx_all = jnp.stack(
        [x.reshape(N // CHUNK, CHUNK).astype(jnp.int32)
         for x in idx_in[:NUM_SC]])
    tab_all = jnp.stack(
        [jnp.tile(jnp.pad(w, ((0, VOCAB_PAD - VOCAB), (0, 0))), (NW, 1))
         for w in tab_in[:NUM_SC]])
    sc_outs = _sc_lookup(idx_all, tab_all)

    tc_outs = tuple(
        _tc_lookup(
            jnp.pad(x.astype(jnp.int32), ((0, 0), (0, L_PAD - L))
                    ).reshape(B * L_PAD),
            w)
        for x, w in zip(idx_in[NUM_SC:], tab_in[NUM_SC:]))

    return sc_outs + tc_outs
